# baseline (speedup 1.0000x reference)
import functools

import jax
import jax.numpy as jnp
from jax import lax
from jax.experimental import pallas as pl
from jax.experimental.pallas import tpu as pltpu
from jax.experimental.pallas import tpu_sc as plsc

_H = 128
_BLK = 512
_NW = 32
_EBLK = 128


def _ceil_to(x, m):
    return -(-x // m) * m


_RNG = 12544


def _make_seg_sum(nblk, ndst_pad):
    mesh = plsc.VectorSubcoreMesh(core_axis_name="c", subcore_axis_name="s")
    npass = ndst_pad // _RNG
    rp = _RNG + 128
    zrows = rp // 16
    drows = _RNG // 16

    @functools.partial(
        pl.kernel,
        mesh=mesh,
        out_type=jax.ShapeDtypeStruct((2, ndst_pad, _H), jnp.float32),
        scratch_types=[
            pltpu.VMEM((_EBLK,), jnp.int32),
            pltpu.VMEM((_EBLK,), jnp.int32),
            pltpu.VMEM((_EBLK, _H), jnp.float32),
            pltpu.VMEM_SHARED((rp, _H), jnp.float32),
            pltpu.SemaphoreType.DMA,
        ],
    )
    def k(table, gidx, didx, zeros, out, idx_v, didx_v, rows_v, acc, sem):
        c = lax.axis_index("c")
        s = lax.axis_index("s")
        w = s * 2 + c
        for p in range(npass):
            base = p * _RNG
            pltpu.sync_copy(
                zeros.at[pl.ds(s * zrows, zrows)],
                acc.at[pl.ds(s * zrows, zrows)],
            )
            plsc.subcore_barrier()

            def body(b, carry, base=base):
                pltpu.sync_copy(gidx.at[w, b], idx_v)
                pltpu.sync_copy(didx.at[w, b], didx_v)
                pltpu.async_copy(table.at[idx_v], rows_v, sem).wait()
                for j in range(_EBLK // 16):
                    v = didx_v[pl.ds(j * 16, 16)]
                    ok = (v >= base) & (v < base + _RNG)
                    didx_v[pl.ds(j * 16, 16)] = jnp.where(ok, v - base, _RNG)
                pltpu.sync_copy(rows_v, acc.at[didx_v], add=True)
                return carry

            lax.fori_loop(0, nblk, body, 0)
            plsc.subcore_barrier()
            pltpu.sync_copy(
                acc.at[pl.ds(s * drows, drows)],
                out.at[c, pl.ds(base + s * drows, drows)],
            )
            plsc.subcore_barrier()

    return k


def _edge_plan(src, dst, ndst_pad):
    e = src.shape[0]
    epad = _ceil_to(e, _NW * _EBLK)
    nblk = epad // (_NW * _EBLK)
    src_p = jnp.pad(src.astype(jnp.int32), (0, epad - e))
    dst_p = jnp.pad(
        dst.astype(jnp.int32), (0, epad - e), constant_values=ndst_pad - 1
    )
    didx = dst_p.reshape(_NW, nblk, _EBLK)
    gidx = src_p.reshape(_NW, nblk, _EBLK)
    return gidx, didx, nblk


def _seg_sum_parts(table_f, gidx, didx, nblk, ndst_pad):
    k = _make_seg_sum(nblk, ndst_pad)
    zeros = jnp.zeros((_RNG + 128, _H), jnp.float32)
    return k(table_f, gidx, didx, zeros)


def _seg_counts(didx, nblk, ndst_pad):
    k = _make_seg_sum(nblk, ndst_pad)
    ones_tab = jnp.ones((8, _H), jnp.float32)
    gz = jnp.zeros((_NW, nblk, _EBLK), jnp.int32)
    zeros = jnp.zeros((_RNG + 128, _H), jnp.float32)
    return k(ones_tab, gz, didx, zeros)


def _rowblk(i):
    return (i, 0)


def _at0(i):
    return (0, 0)


def _emb_body(is_comp, xb_ref, tab_ref, o_ref):
    xb = xb_ref[...]
    tab = tab_ref[...]
    iota = lax.broadcasted_iota(jnp.int32, xb.shape, 1)
    oh = (iota == xb[:, 0:1]).astype(jnp.float32)
    if is_comp:
        oh += (iota == 4).astype(jnp.float32)
    else:
        oh += (iota == (jnp.maximum(xb[:, 1:2], 0) + 4)).astype(jnp.float32)
    oh += (iota == (jnp.maximum(xb[:, 2:3], 0) + 13)).astype(jnp.float32)
    o_ref[...] = jnp.dot(oh, tab, preferred_element_type=jnp.float32)


def _embed(x, npad, table, is_comp):
    xp = jnp.pad(x.astype(jnp.int32), ((0, npad - x.shape[0]), (0, _H - 3)))
    grid = npad // _BLK
    return pl.pallas_call(
        functools.partial(_emb_body, is_comp),
        grid=(grid,),
        in_specs=[
            pl.BlockSpec((_BLK, _H), _rowblk),
            pl.BlockSpec((_H, _H), _at0),
        ],
        out_specs=pl.BlockSpec((_BLK, _H), _rowblk),
        out_shape=jax.ShapeDtypeStruct((npad, _H), jnp.float32),
    )(xp, table)


def _mean(p_ref, c_ref):
    cnt = jnp.max(c_ref[0] + c_ref[1], axis=1, keepdims=True)
    return (p_ref[0] + p_ref[1]) / jnp.maximum(cnt, 1.0)


def _pin_l1_body(x_ref, tab_ref, pcp, ccp, psp, csp, pnp, cnp,
                 wcl, wcr, bc, wsl, wsr, bs, wnl, wnr, bn, o_ref):
    def emb():
        xb = x_ref[...]
        iota = lax.broadcasted_iota(jnp.int32, xb.shape, 1)
        oh = (iota == xb[:, 0:1]).astype(jnp.float32)
        oh += (iota == (jnp.maximum(xb[:, 1:2], 0) + 4)).astype(jnp.float32)
        oh += (iota == (jnp.maximum(xb[:, 2:3], 0) + 13)).astype(jnp.float32)
        return jnp.dot(oh, tab_ref[...], preferred_element_type=jnp.float32)

    root = emb()
    h = jnp.dot(_mean(pcp, ccp), wcl[...], preferred_element_type=jnp.float32)
    h += bc[...] + jnp.dot(root, wcr[...], preferred_element_type=jnp.float32)
    h += jnp.dot(_mean(psp, csp), wsl[...], preferred_element_type=jnp.float32)
    h += bs[...] + jnp.dot(root, wsr[...], preferred_element_type=jnp.float32)
    h += jnp.dot(_mean(pnp, cnp), wnl[...], preferred_element_type=jnp.float32)
    h += bn[...] + jnp.dot(root, wnr[...], preferred_element_type=jnp.float32)
    o_ref[...] = jnp.maximum(h, 0.0)


def _comp_l12_body(root_ref, p1, c1, w1l, w1r, b1, p2, c2, w2l, w2r, b2,
                   clsw, clsb, o_ref):
    h1 = jnp.dot(_mean(p1, c1), w1l[...], preferred_element_type=jnp.float32)
    h1 += b1[...] + jnp.dot(root_ref[...], w1r[...],
                            preferred_element_type=jnp.float32)
    h1 = jnp.maximum(h1, 0.0)
    h2 = jnp.dot(_mean(p2, c2), w2l[...], preferred_element_type=jnp.float32)
    h2 += b2[...] + jnp.dot(h1, w2r[...], preferred_element_type=jnp.float32)
    h2 = jnp.maximum(h2, 0.0)
    o_ref[...] = (
        jnp.dot(h2, clsw[...], preferred_element_type=jnp.float32) + clsb[...]
    )


def kernel(x_component, x_pin, x_subcircuit, x_net, ei_cp, ei_pc, ei_sp,
           ei_ps, ei_pn, ei_np, node_emb, comp_emb, pin_emb, W1_cp_l, W1_cp_r,
           b1_cp, W1_pc_l, W1_pc_r, b1_pc, W1_sp_l, W1_sp_r, b1_sp, W1_ps_l,
           W1_ps_r, b1_ps, W1_pn_l, W1_pn_r, b1_pn, W1_np_l, W1_np_r, b1_np,
           W2_cp_l, W2_cp_r, b2_cp, W2_pc_l, W2_pc_r, b2_pc, W2_sp_l, W2_sp_r,
           b2_sp, W2_ps_l, W2_ps_r, b2_ps, W2_pn_l, W2_pn_r, b2_pn, W2_np_l,
           W2_np_r, b2_np, cls_W, cls_b):
    n_comp, n_pin = x_component.shape[0], x_pin.shape[0]
    npad_c = _ceil_to(n_comp, _BLK)
    npad_p = _ceil_to(n_pin, _BLK)
    npad_s = _ceil_to(x_subcircuit.shape[0], _BLK)
    npad_n = _ceil_to(x_net.shape[0], _BLK)

    tab = jnp.concatenate([node_emb, comp_emb, pin_emb], axis=0)
    tab = jnp.pad(tab, ((0, _H - tab.shape[0]), (0, 0)))

    e_comp = _embed(x_component, npad_c, tab, True)
    e_pin = _embed(x_pin, npad_p, tab, False)
    e_sub = _embed(x_subcircuit, npad_s, tab, False)
    e_net = _embed(x_net, npad_n, tab, False)

    g_cp, d_cp, nb_cp = _edge_plan(ei_cp[0], ei_cp[1], npad_p)
    g_sp, d_sp, nb_sp = _edge_plan(ei_sp[0], ei_sp[1], npad_p)
    g_np, d_np, nb_np = _edge_plan(ei_np[0], ei_np[1], npad_p)
    g_pc, d_pc, nb_pc = _edge_plan(ei_pc[0], ei_pc[1], npad_c)

    p_cp = _seg_sum_parts(e_comp, g_cp, d_cp, nb_cp, npad_p)
    p_sp = _seg_sum_parts(e_sub, g_sp, d_sp, nb_sp, npad_p)
    p_np = _seg_sum_parts(e_net, g_np, d_np, nb_np, npad_p)
    p_pc1 = _seg_sum_parts(e_pin, g_pc, d_pc, nb_pc, npad_c)
    c_cp = _seg_counts(d_cp, nb_cp, npad_p)
    c_sp = _seg_counts(d_sp, nb_sp, npad_p)
    c_np = _seg_counts(d_np, nb_np, npad_p)
    c_pc = _seg_counts(d_pc, nb_pc, npad_c)

    xpp = jnp.pad(x_pin.astype(jnp.int32), ((0, npad_p - n_pin), (0, _H - 3)))
    rb = pl.BlockSpec((_BLK, _H), _rowblk)
    pb = pl.BlockSpec((2, _BLK, _H), lambda i: (0, i, 0))
    cb = pb
    wf = pl.BlockSpec((_H, _H), _at0)
    bf = pl.BlockSpec((1, _H), _at0)

    h1_pin = pl.pallas_call(
        _pin_l1_body,
        grid=(npad_p // _BLK,),
        in_specs=[rb, wf, pb, cb, pb, cb, pb, cb,
                  wf, wf, bf, wf, wf, bf, wf, wf, bf],
        out_specs=rb,
        out_shape=jax.ShapeDtypeStruct((npad_p, _H), jnp.float32),
    )(xpp, tab, p_cp, c_cp, p_sp, c_sp, p_np, c_np,
      W1_cp_l, W1_cp_r, b1_cp.reshape(1, _H),
      W1_sp_l, W1_sp_r, b1_sp.reshape(1, _H),
      W1_np_l, W1_np_r, b1_np.reshape(1, _H))

    p_pc2 = _seg_sum_parts(h1_pin, g_pc, d_pc, nb_pc, npad_c)

    clsw = jnp.pad(cls_W, ((0, 0), (0, _H - cls_W.shape[1])))
    clsb = jnp.pad(cls_b, (0, _H - cls_b.shape[0])).reshape(1, _H)

    out = pl.pallas_call(
        _comp_l12_body,
        grid=(npad_c // _BLK,),
        in_specs=[rb, pb, cb, wf, wf, bf, pb, cb, wf, wf, bf, wf, bf],
        out_specs=rb,
        out_shape=jax.ShapeDtypeStruct((npad_c, _H), jnp.float32),
    )(e_comp, p_pc1, c_pc, W1_pc_l, W1_pc_r, b1_pc.reshape(1, _H),
      p_pc2, c_pc, W2_pc_l, W2_pc_r, b2_pc.reshape(1, _H), clsw, clsb)

    return out[:n_comp, :cls_W.shape[1]]
